# baseline (speedup 1.0000x reference)
import functools

import functools

import jax
import jax.numpy as jnp
from jax import lax
from jax.experimental import pallas as pl
from jax.experimental.pallas import tpu as pltpu
from jax.experimental.pallas import tpu_sc as plsc

_COMMIT = 0.25


_R = 400
_C = 2048


def _dist_argmin_body(xsq_ref, wsq_ref, x_ref, w_ref, out_ref):
    s2 = lax.dot_general(x_ref[...], w_ref[...], (((1,), (1,)), ((), ())),
                         preferred_element_type=jnp.float32)
    d = (xsq_ref[...] + wsq_ref[0]) - s2
    di = lax.bitcast_convert_type(d, jnp.int32)
    rel = di - di[:, 0:1]
    lane = lax.broadcasted_iota(jnp.int32, d.shape, 1)
    key = jnp.left_shift(rel, 13) + lane
    kmin = jnp.min(key, axis=1, keepdims=True)
    out_ref[...] = jnp.bitwise_and(kmin, d.shape[1] - 1)


def _dist_argmin(x, weight, xsq, wsq):
    n, d_model = x.shape
    k = weight.shape[0]
    nblk = n // _R
    wsq3 = wsq.reshape(1, 1, k)
    out = pl.pallas_call(
        _dist_argmin_body,
        grid=(nblk,),
        in_specs=[
            pl.BlockSpec((_R, 1), lambda i: (i, 0)),
            pl.BlockSpec((1, 1, k), lambda i: (0, 0, 0)),
            pl.BlockSpec((_R, d_model), lambda i: (i, 0)),
            pl.BlockSpec((k, d_model), lambda i: (0, 0)),
        ],
        out_specs=pl.BlockSpec((_R, 1), lambda i: (i, 0)),
        out_shape=jax.ShapeDtypeStruct((n, 1), jnp.int32),
    )(xsq, wsq3, 2.0 * x, weight)
    return out.reshape(n)


_NW = 32
_BPW = 320
_SUB = 80
_NSUB = _BPW // _SUB


def _sc_stage2_body(w_hbm, idx_hbm, x_hbm, out_hbm, part_hbm,
                    idx_v, rows_v, x_v, acc_v, sem):
    d_model = w_hbm.shape[1]
    nl = d_model // 16
    wid = lax.axis_index("s") * 2 + lax.axis_index("c")
    base = wid * _BPW
    acc = jnp.zeros((16,), jnp.float32)
    for cc in range(_NSUB):
        start = base + cc * _SUB
        pltpu.sync_copy(idx_hbm.at[pl.ds(start, _SUB)], idx_v)
        pltpu.async_copy(w_hbm.at[idx_v], rows_v, sem).wait()
        pltpu.sync_copy(x_hbm.at[pl.ds(start, _SUB)], x_v)

        def row_body(r, a):
            def col_body(q, a2):
                sl = pl.ds(q * 16, 16)
                zq = rows_v[r, sl]
                xx = x_v[r, sl]
                t = zq - xx
                rows_v[r, sl] = xx + t
                return a2 + t * t
            return lax.fori_loop(0, nl, col_body, a)

        acc = lax.fori_loop(0, _SUB, row_body, acc)
        pltpu.sync_copy(rows_v, out_hbm.at[pl.ds(start, _SUB)])
    acc_v[...] = acc
    pltpu.sync_copy(acc_v, part_hbm.at[wid])


def _sc_stage2(weight, idx_p, x_p):
    d_model = weight.shape[1]
    np_rows = idx_p.shape[0]
    mesh = plsc.VectorSubcoreMesh(core_axis_name="c", subcore_axis_name="s")
    fn = pl.kernel(
        _sc_stage2_body, mesh=mesh,
        out_type=[jax.ShapeDtypeStruct((np_rows, d_model), jnp.float32),
                  jax.ShapeDtypeStruct((_NW, 16), jnp.float32)],
        scratch_types=[
            pltpu.VMEM((_SUB,), jnp.int32),
            pltpu.VMEM((_SUB, d_model), jnp.float32),
            pltpu.VMEM((_SUB, d_model), jnp.float32),
            pltpu.VMEM((16,), jnp.float32),
            pltpu.SemaphoreType.DMA,
        ],
    )
    return fn(weight, idx_p, x_p)


def kernel(x, edge_index, weight):
    n, d_model = x.shape
    xsq = jnp.sum(x ** 2, axis=1, keepdims=True)
    wsq = jnp.sum(weight ** 2, axis=1)
    idx = _dist_argmin(x, weight, xsq, wsq)

    np_rows = _NW * _BPW
    pad = np_rows - n
    idx_p = jnp.concatenate([idx, jnp.zeros((pad,), jnp.int32)])
    x_p = jnp.concatenate([x, jnp.zeros((pad, d_model), jnp.float32)])
    z_q_st_p, partials = _sc_stage2(weight, idx_p, x_p)
    z_q_st = z_q_st_p[:n]
    m = jnp.sum(partials) / (n * d_model)
    loss = m + _COMMIT * m
    return (z_q_st, edge_index, loss, idx)


# speedup vs baseline: 1.1550x; 1.1550x over previous
import functools

import functools

import jax
import jax.numpy as jnp
from jax import lax
from jax.experimental import pallas as pl
from jax.experimental.pallas import tpu as pltpu
from jax.experimental.pallas import tpu_sc as plsc

_COMMIT = 0.25


_R = 400
_C = 2048


def _dist_argmin_body(xsq_ref, wsq_ref, x_ref, w_ref, out_ref):
    s2 = lax.dot_general(x_ref[...], w_ref[...], (((1,), (1,)), ((), ())),
                         preferred_element_type=jnp.float32)
    d = (xsq_ref[...] + wsq_ref[0]) - s2
    di = lax.bitcast_convert_type(d, jnp.int32)
    rel = di - di[:, 0:1]
    lane = lax.broadcasted_iota(jnp.int32, d.shape, 1)
    key = jnp.left_shift(rel, 13) + lane
    kmin = jnp.min(key, axis=1, keepdims=True)
    out_ref[...] = jnp.bitwise_and(kmin, d.shape[1] - 1)


def _dist_argmin(x, weight, xsq, wsq):
    n, d_model = x.shape
    k = weight.shape[0]
    nblk = n // _R
    wsq3 = wsq.reshape(1, 1, k)
    out = pl.pallas_call(
        _dist_argmin_body,
        grid=(nblk,),
        in_specs=[
            pl.BlockSpec((_R, 1), lambda i: (i, 0)),
            pl.BlockSpec((1, 1, k), lambda i: (0, 0, 0)),
            pl.BlockSpec((_R, d_model), lambda i: (i, 0)),
            pl.BlockSpec((k, d_model), lambda i: (0, 0)),
        ],
        out_specs=pl.BlockSpec((_R, 1), lambda i: (i, 0)),
        out_shape=jax.ShapeDtypeStruct((n, 1), jnp.int32),
    )(xsq, wsq3, 2.0 * x, weight)
    return out.reshape(n)


_NW = 32
_BPW = 320
_SUB = 80
_NSUB = _BPW // _SUB


def _sc_stage2_body(w_hbm, idx_hbm, x_hbm, out_hbm, part_hbm,
                    idx0, idx1, rows0, rows1, x0, x1, acc_v,
                    g0, g1, xs0, xs1, os0, os1):
    d_model = w_hbm.shape[1]
    nq = d_model // 16
    wid = lax.axis_index("s") * 2 + lax.axis_index("c")
    base = wid * _BPW
    bufs = [(idx0, rows0, x0, g0, xs0, os0),
            (idx1, rows1, x1, g1, xs1, os1)]

    def start_fetch(cc):
        idx_v, rows_v, x_v, gsem, xsem, _ = bufs[cc % 2]
        st = base + cc * _SUB
        pltpu.sync_copy(idx_hbm.at[pl.ds(st, _SUB)], idx_v)
        gh = pltpu.async_copy(w_hbm.at[idx_v], rows_v, gsem)
        xh = pltpu.async_copy(x_hbm.at[pl.ds(st, _SUB)], x_v, xsem)
        return gh, xh

    fetch_h = {0: start_fetch(0)}
    out_h = {}
    accs = [jnp.zeros((16,), jnp.float32) for _ in range(4)]
    for cc in range(_NSUB):
        idx_v, rows_v, x_v, _, _, osem = bufs[cc % 2]
        if cc + 1 < _NSUB:
            if cc >= 1:
                out_h.pop(cc - 1).wait()
            fetch_h[cc + 1] = start_fetch(cc + 1)
        gh, xh = fetch_h.pop(cc)
        gh.wait()
        xh.wait()

        def row_body(r, a):
            a = list(a)
            for q in range(nq):
                sl = pl.ds(q * 16, 16)
                zq = rows_v[r, sl]
                xx = x_v[r, sl]
                t = zq - xx
                rows_v[r, sl] = xx + t
                a[q % 4] = a[q % 4] + t * t
            return tuple(a)

        accs = lax.fori_loop(0, _SUB, row_body, tuple(accs))
        out_h[cc] = pltpu.async_copy(
            rows_v, out_hbm.at[pl.ds(base + cc * _SUB, _SUB)], osem)
    out_h.pop(_NSUB - 2).wait()
    out_h.pop(_NSUB - 1).wait()
    acc_v[...] = (accs[0] + accs[1]) + (accs[2] + accs[3])
    pltpu.sync_copy(acc_v, part_hbm.at[wid])


def _sc_stage2(weight, idx_p, x_p):
    d_model = weight.shape[1]
    np_rows = idx_p.shape[0]
    mesh = plsc.VectorSubcoreMesh(core_axis_name="c", subcore_axis_name="s")
    fn = pl.kernel(
        _sc_stage2_body, mesh=mesh,
        out_type=[jax.ShapeDtypeStruct((np_rows, d_model), jnp.float32),
                  jax.ShapeDtypeStruct((_NW, 16), jnp.float32)],
        scratch_types=[
            pltpu.VMEM((_SUB,), jnp.int32),
            pltpu.VMEM((_SUB,), jnp.int32),
            pltpu.VMEM((_SUB, d_model), jnp.float32),
            pltpu.VMEM((_SUB, d_model), jnp.float32),
            pltpu.VMEM((_SUB, d_model), jnp.float32),
            pltpu.VMEM((_SUB, d_model), jnp.float32),
            pltpu.VMEM((16,), jnp.float32),
            pltpu.SemaphoreType.DMA,
            pltpu.SemaphoreType.DMA,
            pltpu.SemaphoreType.DMA,
            pltpu.SemaphoreType.DMA,
            pltpu.SemaphoreType.DMA,
            pltpu.SemaphoreType.DMA,
        ],
    )
    return fn(weight, idx_p, x_p)


def kernel(x, edge_index, weight):
    n, d_model = x.shape
    xsq = jnp.sum(x ** 2, axis=1, keepdims=True)
    wsq = jnp.sum(weight ** 2, axis=1)
    idx = _dist_argmin(x, weight, xsq, wsq)

    np_rows = _NW * _BPW
    pad = np_rows - n
    idx_p = jnp.concatenate([idx, jnp.zeros((pad,), jnp.int32)])
    x_p = jnp.concatenate([x, jnp.zeros((pad, d_model), jnp.float32)])
    z_q_st_p, partials = _sc_stage2(weight, idx_p, x_p)
    z_q_st = z_q_st_p[:n]
    m = jnp.sum(partials) / (n * d_model)
    loss = m + _COMMIT * m
    return (z_q_st, edge_index, loss, idx)


# speedup vs baseline: 1.3268x; 1.1487x over previous
import functools

import functools

import jax
import jax.numpy as jnp
from jax import lax
from jax.experimental import pallas as pl
from jax.experimental.pallas import tpu as pltpu
from jax.experimental.pallas import tpu_sc as plsc

_COMMIT = 0.25


_R = 400
_C = 2048


def _dist_argmin_body(wsq_ref, x_ref, w_ref, out_ref):
    xb = x_ref[...]
    xsq = jnp.sum(xb ** 2, axis=1, keepdims=True)
    s = lax.dot_general(xb, w_ref[...], (((1,), (1,)), ((), ())),
                        preferred_element_type=jnp.float32)
    d = (xsq + wsq_ref[0]) - 2.0 * s
    di = lax.bitcast_convert_type(d, jnp.int32)
    rel = di - di[:, 0:1]
    lane = lax.broadcasted_iota(jnp.int32, d.shape, 1)
    key = jnp.left_shift(rel, 13) + lane
    kmin = jnp.min(key, axis=1, keepdims=True)
    out_ref[...] = jnp.bitwise_and(kmin, d.shape[1] - 1)


def _dist_argmin(x, weight, wsq):
    n, d_model = x.shape
    k = weight.shape[0]
    nblk = n // _R
    wsq3 = wsq.reshape(1, 1, k)
    out = pl.pallas_call(
        _dist_argmin_body,
        grid=(nblk,),
        in_specs=[
            pl.BlockSpec((1, 1, k), lambda i: (0, 0, 0)),
            pl.BlockSpec((_R, d_model), lambda i: (i, 0)),
            pl.BlockSpec((k, d_model), lambda i: (0, 0)),
        ],
        out_specs=pl.BlockSpec((_R, 1), lambda i: (i, 0)),
        out_shape=jax.ShapeDtypeStruct((n, 1), jnp.int32),
    )(wsq3, x, weight)
    return out.reshape(n)


_NW = 32
_BPW = 312
_SUB = 104
_NSUB = _BPW // _SUB
_TAIL = 16


def _row_loop(rows_v, x_v, nrows, nq, init):
    def row_body(r, a):
        a = list(a)
        for q in range(nq):
            sl = pl.ds(q * 16, 16)
            zq = rows_v[r, sl]
            xx = x_v[r, sl]
            t = zq - xx
            rows_v[r, sl] = xx + t
            a[q % 4] = a[q % 4] + t * t
        return tuple(a)
    return lax.fori_loop(0, nrows, row_body, init)


def _sc_stage2_body(w_hbm, idx_hbm, x_hbm, out_hbm, part_hbm,
                    idx0, idx1, rows0, rows1, x0, x1,
                    tidx, trows, tx, acc_v,
                    g0, g1, xs0, xs1, os0, os1, tsem):
    d_model = w_hbm.shape[1]
    nq = d_model // 16
    wid = lax.axis_index("s") * 2 + lax.axis_index("c")
    base = wid * _BPW
    zero4 = tuple(jnp.zeros((16,), jnp.float32) for _ in range(4))
    acc_v[...] = jnp.zeros((16,), jnp.float32)

    @pl.when(wid == 0)
    def _():
        st = _NW * _BPW
        pltpu.sync_copy(idx_hbm.at[pl.ds(st, _TAIL)], tidx)
        pltpu.async_copy(w_hbm.at[tidx], trows, tsem).wait()
        pltpu.sync_copy(x_hbm.at[pl.ds(st, _TAIL)], tx)
        ta = _row_loop(trows, tx, _TAIL, nq, zero4)
        acc_v[...] = (ta[0] + ta[1]) + (ta[2] + ta[3])
        pltpu.sync_copy(trows, out_hbm.at[pl.ds(st, _TAIL)])

    bufs = [(idx0, rows0, x0, g0, xs0, os0),
            (idx1, rows1, x1, g1, xs1, os1)]

    def start_fetch(cc):
        idx_v, rows_v, x_v, gsem, xsem, _ = bufs[cc % 2]
        st = base + cc * _SUB
        pltpu.sync_copy(idx_hbm.at[pl.ds(st, _SUB)], idx_v)
        gh = pltpu.async_copy(w_hbm.at[idx_v], rows_v, gsem)
        xh = pltpu.async_copy(x_hbm.at[pl.ds(st, _SUB)], x_v, xsem)
        return gh, xh

    fetch_h = {0: start_fetch(0)}
    out_h = {}
    accs = zero4
    for cc in range(_NSUB):
        idx_v, rows_v, x_v, _, _, osem = bufs[cc % 2]
        if cc + 1 < _NSUB:
            if cc >= 1:
                out_h.pop(cc - 1).wait()
            fetch_h[cc + 1] = start_fetch(cc + 1)
        gh, xh = fetch_h.pop(cc)
        gh.wait()
        xh.wait()
        accs = _row_loop(rows_v, x_v, _SUB, nq, accs)
        out_h[cc] = pltpu.async_copy(
            rows_v, out_hbm.at[pl.ds(base + cc * _SUB, _SUB)], osem)
    for cc in sorted(out_h):
        out_h[cc].wait()
    acc_v[...] = acc_v[...] + ((accs[0] + accs[1]) + (accs[2] + accs[3]))
    pltpu.sync_copy(acc_v, part_hbm.at[wid])


def _sc_stage2(weight, idx, x):
    d_model = weight.shape[1]
    n = idx.shape[0]
    mesh = plsc.VectorSubcoreMesh(core_axis_name="c", subcore_axis_name="s")
    fn = pl.kernel(
        _sc_stage2_body, mesh=mesh,
        out_type=[jax.ShapeDtypeStruct((n, d_model), jnp.float32),
                  jax.ShapeDtypeStruct((_NW, 16), jnp.float32)],
        scratch_types=[
            pltpu.VMEM((_SUB,), jnp.int32),
            pltpu.VMEM((_SUB,), jnp.int32),
            pltpu.VMEM((_SUB, d_model), jnp.float32),
            pltpu.VMEM((_SUB, d_model), jnp.float32),
            pltpu.VMEM((_SUB, d_model), jnp.float32),
            pltpu.VMEM((_SUB, d_model), jnp.float32),
            pltpu.VMEM((_TAIL,), jnp.int32),
            pltpu.VMEM((_TAIL, d_model), jnp.float32),
            pltpu.VMEM((_TAIL, d_model), jnp.float32),
            pltpu.VMEM((16,), jnp.float32),
            pltpu.SemaphoreType.DMA,
            pltpu.SemaphoreType.DMA,
            pltpu.SemaphoreType.DMA,
            pltpu.SemaphoreType.DMA,
            pltpu.SemaphoreType.DMA,
            pltpu.SemaphoreType.DMA,
            pltpu.SemaphoreType.DMA,
        ],
    )
    return fn(weight, idx, x)


def kernel(x, edge_index, weight):
    n, d_model = x.shape
    wsq = jnp.sum(weight ** 2, axis=1)
    idx = _dist_argmin(x, weight, wsq)

    z_q_st, partials = _sc_stage2(weight, idx, x)
    m = jnp.sum(partials) / (n * d_model)
    loss = m + _COMMIT * m
    return (z_q_st, edge_index, loss, idx)


# speedup vs baseline: 1.3305x; 1.0028x over previous
import functools

import functools

import jax
import jax.numpy as jnp
from jax import lax
from jax.experimental import pallas as pl
from jax.experimental.pallas import tpu as pltpu
from jax.experimental.pallas import tpu_sc as plsc

_COMMIT = 0.25


_R = 400
_C = 2048


def _dist_argmin_body(wsq_ref, x_ref, w_ref, out_ref):
    xb = x_ref[...]
    xsq = 0.25 * jnp.sum(xb * xb, axis=1, keepdims=True)
    s2 = lax.dot_general(xb, w_ref[...], (((1,), (1,)), ((), ())),
                         preferred_element_type=jnp.float32)
    d = (xsq + wsq_ref[0]) - s2
    di = lax.bitcast_convert_type(d, jnp.int32)
    rel = di - di[:, 0:1]
    lane = lax.broadcasted_iota(jnp.int32, d.shape, 1)
    key = jnp.left_shift(rel, 13) + lane
    kmin = jnp.min(key, axis=1, keepdims=True)
    out_ref[...] = jnp.bitwise_and(kmin, d.shape[1] - 1)


def _dist_argmin(x, weight, wsq):
    n, d_model = x.shape
    k = weight.shape[0]
    nblk = n // _R
    wsq3 = wsq.reshape(1, 1, k)
    out = pl.pallas_call(
        _dist_argmin_body,
        grid=(nblk,),
        in_specs=[
            pl.BlockSpec((1, 1, k), lambda i: (0, 0, 0)),
            pl.BlockSpec((_R, d_model), lambda i: (i, 0)),
            pl.BlockSpec((k, d_model), lambda i: (0, 0)),
        ],
        out_specs=pl.BlockSpec((_R, 1), lambda i: (i, 0)),
        out_shape=jax.ShapeDtypeStruct((n, 1), jnp.int32),
    )(wsq3, 2.0 * x, weight)
    return out.reshape(n)


_NW = 32
_BPW = 312
_SUB = 104
_NSUB = _BPW // _SUB
_TAIL = 16


def _row_loop(rows_v, x_v, nrows, nq, init):
    def row_body(r, a):
        a = list(a)
        for q in range(nq):
            sl = pl.ds(q * 16, 16)
            zq = rows_v[r, sl]
            xx = x_v[r, sl]
            t = zq - xx
            rows_v[r, sl] = xx + t
            a[q % 4] = a[q % 4] + t * t
        return tuple(a)
    return lax.fori_loop(0, nrows, row_body, init)


def _sc_stage2_body(w_hbm, idx_hbm, x_hbm, out_hbm, part_hbm,
                    idx0, idx1, rows0, rows1, x0, x1,
                    tidx, trows, tx, acc_v,
                    g0, g1, xs0, xs1, os0, os1, tsem):
    d_model = w_hbm.shape[1]
    nq = d_model // 16
    wid = lax.axis_index("s") * 2 + lax.axis_index("c")
    base = wid * _BPW
    zero4 = tuple(jnp.zeros((16,), jnp.float32) for _ in range(4))
    acc_v[...] = jnp.zeros((16,), jnp.float32)

    @pl.when(wid == 0)
    def _():
        st = _NW * _BPW
        pltpu.sync_copy(idx_hbm.at[pl.ds(st, _TAIL)], tidx)
        pltpu.async_copy(w_hbm.at[tidx], trows, tsem).wait()
        pltpu.sync_copy(x_hbm.at[pl.ds(st, _TAIL)], tx)
        ta = _row_loop(trows, tx, _TAIL, nq, zero4)
        acc_v[...] = (ta[0] + ta[1]) + (ta[2] + ta[3])
        pltpu.sync_copy(trows, out_hbm.at[pl.ds(st, _TAIL)])

    bufs = [(idx0, rows0, x0, g0, xs0, os0),
            (idx1, rows1, x1, g1, xs1, os1)]

    def start_fetch(cc):
        idx_v, rows_v, x_v, gsem, xsem, _ = bufs[cc % 2]
        st = base + cc * _SUB
        pltpu.sync_copy(idx_hbm.at[pl.ds(st, _SUB)], idx_v)
        gh = pltpu.async_copy(w_hbm.at[idx_v], rows_v, gsem)
        xh = pltpu.async_copy(x_hbm.at[pl.ds(st, _SUB)], x_v, xsem)
        return gh, xh

    fetch_h = {0: start_fetch(0)}
    out_h = {}
    accs = zero4
    for cc in range(_NSUB):
        idx_v, rows_v, x_v, _, _, osem = bufs[cc % 2]
        if cc + 1 < _NSUB:
            if cc >= 1:
                out_h.pop(cc - 1).wait()
            fetch_h[cc + 1] = start_fetch(cc + 1)
        gh, xh = fetch_h.pop(cc)
        gh.wait()
        xh.wait()
        accs = _row_loop(rows_v, x_v, _SUB, nq, accs)
        out_h[cc] = pltpu.async_copy(
            rows_v, out_hbm.at[pl.ds(base + cc * _SUB, _SUB)], osem)
    for cc in sorted(out_h):
        out_h[cc].wait()
    acc_v[...] = acc_v[...] + ((accs[0] + accs[1]) + (accs[2] + accs[3]))
    pltpu.sync_copy(acc_v, part_hbm.at[wid])


def _sc_stage2(weight, idx, x):
    d_model = weight.shape[1]
    n = idx.shape[0]
    mesh = plsc.VectorSubcoreMesh(core_axis_name="c", subcore_axis_name="s")
    fn = pl.kernel(
        _sc_stage2_body, mesh=mesh,
        out_type=[jax.ShapeDtypeStruct((n, d_model), jnp.float32),
                  jax.ShapeDtypeStruct((_NW, 16), jnp.float32)],
        scratch_types=[
            pltpu.VMEM((_SUB,), jnp.int32),
            pltpu.VMEM((_SUB,), jnp.int32),
            pltpu.VMEM((_SUB, d_model), jnp.float32),
            pltpu.VMEM((_SUB, d_model), jnp.float32),
            pltpu.VMEM((_SUB, d_model), jnp.float32),
            pltpu.VMEM((_SUB, d_model), jnp.float32),
            pltpu.VMEM((_TAIL,), jnp.int32),
            pltpu.VMEM((_TAIL, d_model), jnp.float32),
            pltpu.VMEM((_TAIL, d_model), jnp.float32),
            pltpu.VMEM((16,), jnp.float32),
            pltpu.SemaphoreType.DMA,
            pltpu.SemaphoreType.DMA,
            pltpu.SemaphoreType.DMA,
            pltpu.SemaphoreType.DMA,
            pltpu.SemaphoreType.DMA,
            pltpu.SemaphoreType.DMA,
            pltpu.SemaphoreType.DMA,
        ],
    )
    return fn(weight, idx, x)


def kernel(x, edge_index, weight):
    n, d_model = x.shape
    wsq = jnp.sum(weight ** 2, axis=1)
    idx = _dist_argmin(x, weight, wsq)

    z_q_st, partials = _sc_stage2(weight, idx, x)
    m = jnp.sum(partials) / (n * d_model)
    loss = m + _COMMIT * m
    return (z_q_st, edge_index, loss, idx)


# speedup vs baseline: 1.3452x; 1.0110x over previous
import functools

import functools

import jax
import jax.numpy as jnp
from jax import lax
from jax.experimental import pallas as pl
from jax.experimental.pallas import tpu as pltpu
from jax.experimental.pallas import tpu_sc as plsc

_COMMIT = 0.25


_R = 400
_C = 2048


def _dist_argmin_body(x_ref, w_ref, out_ref, wsq_ref):
    @pl.when(pl.program_id(0) == 0)
    def _():
        wb = w_ref[...]
        wsq_ref[...] = jnp.sum(wb * wb, axis=1).reshape(1, wb.shape[0])

    xb = x_ref[...]
    xsq = 0.25 * jnp.sum(xb * xb, axis=1, keepdims=True)
    s2 = lax.dot_general(xb, w_ref[...], (((1,), (1,)), ((), ())),
                         preferred_element_type=jnp.float32)
    d = (xsq + wsq_ref[...]) - s2
    di = lax.bitcast_convert_type(d, jnp.int32)
    rel = di - di[:, 0:1]
    lane = lax.broadcasted_iota(jnp.int32, d.shape, 1)
    key = jnp.left_shift(rel, 13) + lane
    kmin = jnp.min(key, axis=1, keepdims=True)
    out_ref[...] = jnp.bitwise_and(kmin, d.shape[1] - 1)


def _dist_argmin(x, weight):
    n, d_model = x.shape
    k = weight.shape[0]
    nblk = n // _R
    out = pl.pallas_call(
        _dist_argmin_body,
        grid=(nblk,),
        in_specs=[
            pl.BlockSpec((_R, d_model), lambda i: (i, 0)),
            pl.BlockSpec((k, d_model), lambda i: (0, 0)),
        ],
        out_specs=pl.BlockSpec((_R, 1), lambda i: (i, 0)),
        out_shape=jax.ShapeDtypeStruct((n, 1), jnp.int32),
        scratch_shapes=[pltpu.VMEM((1, k), jnp.float32)],
    )(2.0 * x, weight)
    return out.reshape(n)


_NW = 32
_BPW = 312
_SUB = 104
_NSUB = _BPW // _SUB
_TAIL = 16


def _row_loop(rows_v, x_v, nrows, nq, init):
    def row_body(r, a):
        a = list(a)
        for q in range(nq):
            sl = pl.ds(q * 16, 16)
            zq = rows_v[r, sl]
            xx = x_v[r, sl]
            t = zq - xx
            rows_v[r, sl] = xx + t
            a[q % 4] = a[q % 4] + t * t
        return tuple(a)
    return lax.fori_loop(0, nrows, row_body, init)


def _sc_stage2_body(w_hbm, idx_hbm, x_hbm, out_hbm, part_hbm,
                    idx0, idx1, rows0, rows1, x0, x1,
                    tidx, trows, tx, acc_v,
                    g0, g1, xs0, xs1, os0, os1, tsem):
    d_model = w_hbm.shape[1]
    nq = d_model // 16
    wid = lax.axis_index("s") * 2 + lax.axis_index("c")
    base = wid * _BPW
    zero4 = tuple(jnp.zeros((16,), jnp.float32) for _ in range(4))
    acc_v[...] = jnp.zeros((16,), jnp.float32)

    @pl.when(wid == 0)
    def _():
        st = _NW * _BPW
        pltpu.sync_copy(idx_hbm.at[pl.ds(st, _TAIL)], tidx)
        pltpu.async_copy(w_hbm.at[tidx], trows, tsem).wait()
        pltpu.sync_copy(x_hbm.at[pl.ds(st, _TAIL)], tx)
        ta = _row_loop(trows, tx, _TAIL, nq, zero4)
        acc_v[...] = (ta[0] + ta[1]) + (ta[2] + ta[3])
        pltpu.sync_copy(trows, out_hbm.at[pl.ds(st, _TAIL)])

    bufs = [(idx0, rows0, x0, g0, xs0, os0),
            (idx1, rows1, x1, g1, xs1, os1)]

    def start_fetch(cc):
        idx_v, rows_v, x_v, gsem, xsem, _ = bufs[cc % 2]
        st = base + cc * _SUB
        pltpu.sync_copy(idx_hbm.at[pl.ds(st, _SUB)], idx_v)
        gh = pltpu.async_copy(w_hbm.at[idx_v], rows_v, gsem)
        xh = pltpu.async_copy(x_hbm.at[pl.ds(st, _SUB)], x_v, xsem)
        return gh, xh

    fetch_h = {0: start_fetch(0)}
    out_h = {}
    accs = zero4
    for cc in range(_NSUB):
        idx_v, rows_v, x_v, _, _, osem = bufs[cc % 2]
        if cc + 1 < _NSUB:
            if cc >= 1:
                out_h.pop(cc - 1).wait()
            fetch_h[cc + 1] = start_fetch(cc + 1)
        gh, xh = fetch_h.pop(cc)
        gh.wait()
        xh.wait()
        accs = _row_loop(rows_v, x_v, _SUB, nq, accs)
        out_h[cc] = pltpu.async_copy(
            rows_v, out_hbm.at[pl.ds(base + cc * _SUB, _SUB)], osem)
    for cc in sorted(out_h):
        out_h[cc].wait()
    acc_v[...] = acc_v[...] + ((accs[0] + accs[1]) + (accs[2] + accs[3]))
    pltpu.sync_copy(acc_v, part_hbm.at[wid])


def _sc_stage2(weight, idx, x):
    d_model = weight.shape[1]
    n = idx.shape[0]
    mesh = plsc.VectorSubcoreMesh(core_axis_name="c", subcore_axis_name="s")
    fn = pl.kernel(
        _sc_stage2_body, mesh=mesh,
        out_type=[jax.ShapeDtypeStruct((n, d_model), jnp.float32),
                  jax.ShapeDtypeStruct((_NW, 16), jnp.float32)],
        scratch_types=[
            pltpu.VMEM((_SUB,), jnp.int32),
            pltpu.VMEM((_SUB,), jnp.int32),
            pltpu.VMEM((_SUB, d_model), jnp.float32),
            pltpu.VMEM((_SUB, d_model), jnp.float32),
            pltpu.VMEM((_SUB, d_model), jnp.float32),
            pltpu.VMEM((_SUB, d_model), jnp.float32),
            pltpu.VMEM((_TAIL,), jnp.int32),
            pltpu.VMEM((_TAIL, d_model), jnp.float32),
            pltpu.VMEM((_TAIL, d_model), jnp.float32),
            pltpu.VMEM((16,), jnp.float32),
            pltpu.SemaphoreType.DMA,
            pltpu.SemaphoreType.DMA,
            pltpu.SemaphoreType.DMA,
            pltpu.SemaphoreType.DMA,
            pltpu.SemaphoreType.DMA,
            pltpu.SemaphoreType.DMA,
            pltpu.SemaphoreType.DMA,
        ],
    )
    return fn(weight, idx, x)


def kernel(x, edge_index, weight):
    n, d_model = x.shape
    idx = _dist_argmin(x, weight)

    z_q_st, partials = _sc_stage2(weight, idx, x)
    m = jnp.sum(partials) / (n * d_model)
    loss = m + _COMMIT * m
    return (z_q_st, edge_index, loss, idx)
